# Initial kernel scaffold; baseline (speedup 1.0000x reference)
import jax
import jax.numpy as jnp
from jax.experimental import pallas as pl


def kernel(x, edge_index, edge_attr, params):
    raise NotImplementedError("write your pallas kernel here")


# speedup vs baseline: 1.5970x; 1.5970x over previous
import functools

import numpy as np

import jax
import jax.numpy as jnp
from jax import lax
from jax.experimental import pallas as pl
from jax.experimental.pallas import tpu as pltpu
from jax.experimental.pallas import tpu_sc as plsc

N = 10000
NP = 10240
EG = 320000
F = 64
IN_DIM = 128
EDGE_DIM = 4

BLK = 512
NBLK = NP // BLK

NEG = np.float32(-1e30)
F32_0 = np.float32(0.0)
Z = np.int32(0)


def _layer_prep_body(x_ref, w_ref, as_ref, ad_ref, h_ref, a_ref, amax_ref):
    i = pl.program_id(0)
    h = jnp.dot(x_ref[...], w_ref[...], preferred_element_type=jnp.float32)
    h_ref[...] = h
    a_s = jnp.sum(h * as_ref[...], axis=1)
    a_d = jnp.sum(h * ad_ref[...], axis=1)
    a_ref[...] = jnp.concatenate([a_s[None, :], a_d[None, :]], axis=0)

    rid = lax.broadcasted_iota(jnp.int32, (BLK,), 0) + i * BLK
    valid = rid < N
    cur = jnp.concatenate(
        [jnp.max(jnp.where(valid, a_s, NEG)).reshape(1, 1),
         jnp.max(jnp.where(valid, a_d, NEG)).reshape(1, 1)], axis=1)

    @pl.when(i == 0)
    def _():
        amax_ref[...] = cur

    @pl.when(i > 0)
    def _():
        amax_ref[...] = jnp.maximum(amax_ref[...], cur)


def _layer_prep(x, W, att_src, att_dst, din):
    return pl.pallas_call(
        _layer_prep_body,
        grid=(NBLK,),
        in_specs=[
            pl.BlockSpec((BLK, din), lambda i: (i, Z)),
            pl.BlockSpec((din, F), lambda i: (Z, Z)),
            pl.BlockSpec((1, F), lambda i: (Z, Z)),
            pl.BlockSpec((1, F), lambda i: (Z, Z)),
        ],
        out_specs=[
            pl.BlockSpec((BLK, F), lambda i: (i, Z)),
            pl.BlockSpec((2, BLK), lambda i: (Z, i)),
            pl.BlockSpec((1, 2), lambda i: (Z, Z)),
        ],
        out_shape=[
            jax.ShapeDtypeStruct((NP, F), jnp.float32),
            jax.ShapeDtypeStruct((2, NP), jnp.float32),
            jax.ShapeDtypeStruct((1, 2), jnp.float32),
        ],
    )(x, W, att_src, att_dst)


EBLK = 2560
NEBLK = EG // EBLK


def _edge_b_body(eaT_ref, we_ref, ae_ref, b_ref, bmax_ref):
    i = pl.program_id(0)
    w4 = jnp.dot(we_ref[...], ae_ref[...].T,
                 preferred_element_type=jnp.float32)
    b = jnp.sum(eaT_ref[...] * w4, axis=0, keepdims=True)
    b_ref[...] = b
    cur = jnp.max(b).reshape(1, 1)

    @pl.when(i == 0)
    def _():
        bmax_ref[...] = cur

    @pl.when(i > 0)
    def _():
        bmax_ref[...] = jnp.maximum(bmax_ref[...], cur)


def _edge_b(eaT, W_edge, att_edge):
    return pl.pallas_call(
        _edge_b_body,
        grid=(NEBLK,),
        in_specs=[
            pl.BlockSpec((EDGE_DIM, EBLK), lambda i: (Z, i)),
            pl.BlockSpec((EDGE_DIM, F), lambda i: (Z, Z)),
            pl.BlockSpec((1, F), lambda i: (Z, Z)),
        ],
        out_specs=[
            pl.BlockSpec((1, EBLK), lambda i: (Z, i)),
            pl.BlockSpec((1, 1), lambda i: (Z, Z)),
        ],
        out_shape=[
            jax.ShapeDtypeStruct((1, EG), jnp.float32),
            jax.ShapeDtypeStruct((1, 1), jnp.float32),
        ],
    )(eaT, W_edge, att_edge)


def _loop_b_body(ed_ref, we_ref, ae_ref, b_ref, bmax_ref):
    i = pl.program_id(0)
    w4 = jnp.dot(we_ref[...], ae_ref[...].T,
                 preferred_element_type=jnp.float32)
    eb = ed_ref[0] + ed_ref[1]
    s = jnp.sum(eb[:, 0:EDGE_DIM] * w4[:, 0].reshape(1, EDGE_DIM), axis=1)
    deg = jnp.maximum(eb[:, EDGE_DIM], 1.0)
    b = (s / deg)[None, :]
    b_ref[...] = b
    cur = jnp.max(b).reshape(1, 1)

    @pl.when(i == 0)
    def _():
        bmax_ref[...] = cur

    @pl.when(i > 0)
    def _():
        bmax_ref[...] = jnp.maximum(bmax_ref[...], cur)


def _loop_b(ea_deg, W_edge, att_edge):
    return pl.pallas_call(
        _loop_b_body,
        grid=(NBLK,),
        in_specs=[
            pl.BlockSpec((2, BLK, 8), lambda i: (Z, i, Z)),
            pl.BlockSpec((EDGE_DIM, F), lambda i: (Z, Z)),
            pl.BlockSpec((1, F), lambda i: (Z, Z)),
        ],
        out_specs=[
            pl.BlockSpec((1, BLK), lambda i: (Z, i)),
            pl.BlockSpec((1, 1), lambda i: (Z, Z)),
        ],
        out_shape=[
            jax.ShapeDtypeStruct((1, NP), jnp.float32),
            jax.ShapeDtypeStruct((1, 1), jnp.float32),
        ],
    )(ea_deg, W_edge, att_edge)


def _combine_body(acc_ref, den_ref, bias_ref, g_ref, be_ref, mu_ref, var_ref,
                  out_ref):
    y = acc_ref[0] + acc_ref[1]
    d = den_ref[0, :, 0:1] + den_ref[1, :, 0:1]
    y = y / (d + 1e-16) + bias_ref[...]
    scale = g_ref[...] * lax.rsqrt(var_ref[...] + 1e-5)
    y = scale * (y - mu_ref[...]) + be_ref[...]
    i = pl.program_id(0)
    rid = lax.broadcasted_iota(jnp.int32, (BLK, 1), 0) + i * BLK
    out_ref[...] = jnp.where(rid < N, jnp.maximum(y, F32_0), F32_0)


def _combine(acc, den, bias, gamma, beta, mean, var):
    return pl.pallas_call(
        _combine_body,
        grid=(NBLK,),
        in_specs=[
            pl.BlockSpec((2, BLK, F), lambda i: (Z, i, Z)),
            pl.BlockSpec((2, BLK, 8), lambda i: (Z, i, Z)),
        ] + [pl.BlockSpec((1, F), lambda i: (Z, Z))] * 5,
        out_specs=pl.BlockSpec((BLK, F), lambda i: (i, Z)),
        out_shape=jax.ShapeDtypeStruct((NP, F), jnp.float32),
    )(acc, den, bias, gamma, beta, mean, var)


def _flash_body(xq_ref, xkv_ref, wq_ref, wk_ref, wv_ref, bq_ref, bk_ref,
                bv_ref, wo_ref, bo_ref, w1_ref, b1_ref, w2_ref, b2_ref,
                w3_ref, b3_ref, out_ref,
                macc, mrow, lrow, psum, pmax):
    qi = pl.program_id(0)
    kj = pl.program_id(1)
    nkv = pl.num_programs(1)

    @pl.when(kj == 0)
    def _():
        macc[...] = jnp.zeros_like(macc)
        mrow[...] = jnp.full_like(mrow, NEG)
        lrow[...] = jnp.zeros_like(lrow)

    q = (jnp.dot(xq_ref[...], wq_ref[...], preferred_element_type=jnp.float32)
         + bq_ref[...])
    k = (jnp.dot(xkv_ref[...], wk_ref[...], preferred_element_type=jnp.float32)
         + bk_ref[...])
    v = (jnp.dot(xkv_ref[...], wv_ref[...], preferred_element_type=jnp.float32)
         + bv_ref[...])
    s = lax.dot_general(q, k, (((1,), (1,)), ((), ())),
                        preferred_element_type=jnp.float32) * 0.125
    cid = lax.broadcasted_iota(jnp.int32, (BLK, BLK), 1) + kj * BLK
    s = jnp.where(cid < N, s, NEG)

    m_prev = mrow[...]
    m_new = jnp.maximum(m_prev, jnp.max(s, axis=1, keepdims=True))
    p = jnp.exp(s - m_new[:, 0:1])
    corr = jnp.exp(m_prev - m_new)
    lrow[...] = lrow[...] * corr + jnp.sum(p, axis=1, keepdims=True)
    macc[...] = (macc[...] * corr[:, 0:F]
                 + jnp.dot(p, v, preferred_element_type=jnp.float32))
    mrow[...] = m_new

    @pl.when(kj == nkv - 1)
    def _():
        attn = macc[...] / lrow[:, 0:1]
        y = xq_ref[...] + jnp.dot(attn, wo_ref[...],
                                  preferred_element_type=jnp.float32) + bo_ref[...]
        rid = lax.broadcasted_iota(jnp.int32, (BLK, 1), 0) + qi * BLK
        rvalid = rid < N
        cur_sum = jnp.sum(jnp.where(rvalid, y, F32_0), axis=0, keepdims=True)
        cur_max = jnp.max(jnp.where(rvalid, y, NEG), axis=0, keepdims=True)

        @pl.when(qi == 0)
        def _():
            psum[0:1, :] = cur_sum
            pmax[0:1, :] = cur_max

        @pl.when(qi > 0)
        def _():
            psum[0:1, :] = psum[0:1, :] + cur_sum
            pmax[0:1, :] = jnp.maximum(pmax[0:1, :], cur_max)

        @pl.when(qi == pl.num_programs(0) - 1)
        def _():
            g = jnp.concatenate(
                [psum[0:1, :] * (1.0 / N), pmax[0:1, :]], axis=1)
            h1 = jnp.maximum(
                jnp.dot(g, w1_ref[...], preferred_element_type=jnp.float32)
                + b1_ref[...], 0.0)
            h2 = jnp.maximum(
                jnp.dot(h1, w2_ref[...], preferred_element_type=jnp.float32)
                + b2_ref[...], 0.0)
            out_ref[...] = (jnp.dot(h2, w3_ref[...],
                                    preferred_element_type=jnp.float32)
                            + b3_ref[...])


def _flash(x3, attn_p, mlp_p):
    wq = attn_p["in_w"][0:F].T
    wk = attn_p["in_w"][F:2 * F].T
    wv = attn_p["in_w"][2 * F:].T
    bq = attn_p["in_b"][0:F].reshape(1, F)
    bk = attn_p["in_b"][F:2 * F].reshape(1, F)
    bv = attn_p["in_b"][2 * F:].reshape(1, F)
    wo = attn_p["out_w"].T
    bo = attn_p["out_b"].reshape(1, F)
    w1, b1 = mlp_p["W1"], mlp_p["b1"].reshape(1, -1)
    w2, b2 = mlp_p["W2"], mlp_p["b2"].reshape(1, -1)
    w3, b3 = mlp_p["W3"], mlp_p["b3"].reshape(1, -1)

    const = lambda shape: pl.BlockSpec(shape, lambda i, j: (Z,) * len(shape))
    return pl.pallas_call(
        _flash_body,
        grid=(NBLK, NBLK),
        in_specs=[
            pl.BlockSpec((BLK, F), lambda i, j: (i, Z)),
            pl.BlockSpec((BLK, F), lambda i, j: (j, Z)),
            const((F, F)), const((F, F)), const((F, F)),
            const((1, F)), const((1, F)), const((1, F)),
            const((F, F)), const((1, F)),
            const((2 * F, 128)), const((1, 128)),
            const((128, 64)), const((1, 64)),
            const((64, 192)), const((1, 192)),
        ],
        out_specs=pl.BlockSpec((1, 192), lambda i, j: (Z, Z)),
        out_shape=jax.ShapeDtypeStruct((1, 192), jnp.float32),
        scratch_shapes=[
            pltpu.VMEM((BLK, F), jnp.float32),
            pltpu.VMEM((BLK, 128), jnp.float32),
            pltpu.VMEM((BLK, 128), jnp.float32),
            pltpu.VMEM((8, F), jnp.float32),
            pltpu.VMEM((8, F), jnp.float32),
        ],
    )(x3, x3, wq, wk, wv, bq, bk, bv, wo, bo, w1, b1, w2, b2, w3, b3)


def _sc_deg_easum(dst, eaT):
    ea = eaT.T
    ones = jnp.ones((EG,), jnp.float32)
    acc = jax.ops.segment_sum(
        jnp.concatenate([ea, ones[:, None], jnp.zeros((EG, 3), jnp.float32)],
                        axis=1), dst, num_segments=NP)
    out = jnp.zeros((2, NP, 8), jnp.float32).at[0].set(acc)
    return out


def _sc_spmm(src_f, dst_f, b_all, a_vec, c, h):
    alpha = a_vec[0][src_f] + a_vec[1][dst_f] + b_all
    alpha = jnp.maximum(alpha, 0.2 * alpha)
    ex = jnp.exp(alpha - c)
    acc = jax.ops.segment_sum(ex[:, None] * h[src_f], dst_f, num_segments=NP)
    den = jax.ops.segment_sum(ex, dst_f, num_segments=NP)
    acc_o = jnp.zeros((2, NP, F), jnp.float32).at[0].set(acc)
    den_o = jnp.zeros((2, NP, 8), jnp.float32).at[0, :, 0].set(den)
    return acc_o, den_o


def kernel(x, edge_index, edge_attr, params):
    src = edge_index[0].astype(jnp.int32)
    dst = edge_index[1].astype(jnp.int32)
    loop = jnp.arange(NP, dtype=jnp.int32)
    src_f = jnp.concatenate([src, loop])
    dst_f = jnp.concatenate([dst, loop])
    eaT = edge_attr.T

    ea_deg = _sc_deg_easum(dst, eaT)

    x_cur = jnp.zeros((NP, IN_DIM), jnp.float32).at[:N].set(x)
    din = IN_DIM
    for i in range(3):
        gp = params["gat"][i]
        bp = params["bn"][i]
        h, a_vec, amax = _layer_prep(
            x_cur, gp["W"], gp["att_src"].reshape(1, F),
            gp["att_dst"].reshape(1, F), din)
        b_real, bmax1 = _edge_b(eaT, gp["W_edge"], gp["att_edge"].reshape(1, F))
        b_loop, bmax2 = _loop_b(ea_deg, gp["W_edge"],
                                gp["att_edge"].reshape(1, F))
        b_all = jnp.concatenate([b_real[0], b_loop[0]])
        c = amax[0, 0] + amax[0, 1] + jnp.maximum(bmax1[0, 0], bmax2[0, 0])
        acc, den = _sc_spmm(src_f, dst_f, b_all, a_vec, c, h)
        x_cur = _combine(acc, den, gp["bias"].reshape(1, F), bp["gamma"].reshape(1, F),
                         bp["beta"].reshape(1, F), bp["mean"].reshape(1, F),
                         bp["var"].reshape(1, F))
        din = F

    q = _flash(x_cur, params["attn"], params["mlp"])
    return q.reshape(1, 3, 64)


# speedup vs baseline: 10.4567x; 6.5477x over previous
import functools

import numpy as np

import jax
import jax.numpy as jnp
from jax import lax
from jax.experimental import pallas as pl
from jax.experimental.pallas import tpu as pltpu
from jax.experimental.pallas import tpu_sc as plsc

N = 10000
NP = 10240
EG = 320000
F = 64
IN_DIM = 128
EDGE_DIM = 4

BLK = 512
NBLK = NP // BLK

NEG = np.float32(-1e30)
F32_0 = np.float32(0.0)
Z = np.int32(0)

NC = 2
NS = 16
NTILE = NC * NS
NROW = NP // NS

E_ALL = EG + N
KS = 256
WS = 41
EPAD = NTILE * KS * WS


def _layer_prep_body(x_ref, w_ref, as_ref, ad_ref, h_ref, a_ref, amax_ref):
    i = pl.program_id(0)
    h = jnp.dot(x_ref[...], w_ref[...], preferred_element_type=jnp.float32)
    h_ref[...] = h
    a_s = jnp.sum(h * as_ref[...], axis=1)
    a_d = jnp.sum(h * ad_ref[...], axis=1)
    a_ref[...] = jnp.concatenate([a_s[None, :], a_d[None, :]], axis=0)

    rid = lax.broadcasted_iota(jnp.int32, (BLK,), 0) + i * BLK
    valid = rid < N
    cur = jnp.concatenate(
        [jnp.max(jnp.where(valid, a_s, NEG)).reshape(1, 1),
         jnp.max(jnp.where(valid, a_d, NEG)).reshape(1, 1)], axis=1)

    @pl.when(i == 0)
    def _():
        amax_ref[...] = cur

    @pl.when(i > 0)
    def _():
        amax_ref[...] = jnp.maximum(amax_ref[...], cur)


def _layer_prep(x, W, att_src, att_dst, din):
    return pl.pallas_call(
        _layer_prep_body,
        grid=(NBLK,),
        in_specs=[
            pl.BlockSpec((BLK, din), lambda i: (i, Z)),
            pl.BlockSpec((din, F), lambda i: (Z, Z)),
            pl.BlockSpec((1, F), lambda i: (Z, Z)),
            pl.BlockSpec((1, F), lambda i: (Z, Z)),
        ],
        out_specs=[
            pl.BlockSpec((BLK, F), lambda i: (i, Z)),
            pl.BlockSpec((2, BLK), lambda i: (Z, i)),
            pl.BlockSpec((1, 2), lambda i: (Z, Z)),
        ],
        out_shape=[
            jax.ShapeDtypeStruct((NP, F), jnp.float32),
            jax.ShapeDtypeStruct((2, NP), jnp.float32),
            jax.ShapeDtypeStruct((1, 2), jnp.float32),
        ],
    )(x, W, att_src, att_dst)


EBLK = 2560
NEBLK = EG // EBLK


def _edge_b_body(eaT_ref, we_ref, ae_ref, b_ref, bmax_ref):
    i = pl.program_id(0)
    w4 = jnp.dot(we_ref[...], ae_ref[...].T,
                 preferred_element_type=jnp.float32)
    b = jnp.sum(eaT_ref[...] * w4, axis=0, keepdims=True)
    b_ref[...] = b
    cur = jnp.max(b).reshape(1, 1)

    @pl.when(i == 0)
    def _():
        bmax_ref[...] = cur

    @pl.when(i > 0)
    def _():
        bmax_ref[...] = jnp.maximum(bmax_ref[...], cur)


def _edge_b(eaT, W_edge, att_edge):
    return pl.pallas_call(
        _edge_b_body,
        grid=(NEBLK,),
        in_specs=[
            pl.BlockSpec((EDGE_DIM, EBLK), lambda i: (Z, i)),
            pl.BlockSpec((EDGE_DIM, F), lambda i: (Z, Z)),
            pl.BlockSpec((1, F), lambda i: (Z, Z)),
        ],
        out_specs=[
            pl.BlockSpec((1, EBLK), lambda i: (Z, i)),
            pl.BlockSpec((1, 1), lambda i: (Z, Z)),
        ],
        out_shape=[
            jax.ShapeDtypeStruct((1, EG), jnp.float32),
            jax.ShapeDtypeStruct((1, 1), jnp.float32),
        ],
    )(eaT, W_edge, att_edge)


def _rows8_ea_body(eaT_ref, out_ref):
    ea = eaT_ref[...].T
    ones = jnp.ones((EBLK, 1), jnp.float32)
    zer = jnp.zeros((EBLK, 3), jnp.float32)
    out_ref[...] = jnp.concatenate([ea, ones, zer], axis=1)


def _rows8_ea(eaT):
    return pl.pallas_call(
        _rows8_ea_body,
        grid=(NEBLK,),
        in_specs=[pl.BlockSpec((EDGE_DIM, EBLK), lambda i: (Z, i))],
        out_specs=pl.BlockSpec((EBLK, 8), lambda i: (i, Z)),
        out_shape=jax.ShapeDtypeStruct((EG, 8), jnp.float32),
    )(eaT)


XBLK = 4096
NXBLK = EPAD // XBLK


def _rows8_ex_body(ex_ref, out_ref):
    e = ex_ref[...].reshape(XBLK, 1)
    out_ref[...] = jnp.concatenate(
        [e, jnp.zeros((XBLK, 7), jnp.float32)], axis=1)


def _rows8_ex(ex):
    return pl.pallas_call(
        _rows8_ex_body,
        grid=(NXBLK,),
        in_specs=[pl.BlockSpec((1, XBLK), lambda i: (Z, i))],
        out_specs=pl.BlockSpec((XBLK, 8), lambda i: (i, Z)),
        out_shape=jax.ShapeDtypeStruct((EPAD, 8), jnp.float32),
    )(ex.reshape(1, EPAD))


def _loop_b_body(ed_ref, we_ref, ae_ref, b_ref, bmax_ref):
    i = pl.program_id(0)
    w4 = jnp.dot(we_ref[...], ae_ref[...].T,
                 preferred_element_type=jnp.float32)
    eb = ed_ref[0] + ed_ref[1]
    s = jnp.sum(eb[:, 0:EDGE_DIM] * w4[:, 0].reshape(1, EDGE_DIM), axis=1)
    deg = jnp.maximum(eb[:, EDGE_DIM], 1.0)
    b = (s / deg)[None, :]
    b_ref[...] = b
    cur = jnp.max(b).reshape(1, 1)

    @pl.when(i == 0)
    def _():
        bmax_ref[...] = cur

    @pl.when(i > 0)
    def _():
        bmax_ref[...] = jnp.maximum(bmax_ref[...], cur)


def _loop_b(ea_deg, W_edge, att_edge):
    return pl.pallas_call(
        _loop_b_body,
        grid=(NBLK,),
        in_specs=[
            pl.BlockSpec((2, BLK, 8), lambda i: (Z, i, Z)),
            pl.BlockSpec((EDGE_DIM, F), lambda i: (Z, Z)),
            pl.BlockSpec((1, F), lambda i: (Z, Z)),
        ],
        out_specs=[
            pl.BlockSpec((1, BLK), lambda i: (Z, i)),
            pl.BlockSpec((1, 1), lambda i: (Z, Z)),
        ],
        out_shape=[
            jax.ShapeDtypeStruct((1, NP), jnp.float32),
            jax.ShapeDtypeStruct((1, 1), jnp.float32),
        ],
    )(ea_deg, W_edge, att_edge)


def _combine_body(acc_ref, den_ref, bias_ref, g_ref, be_ref, mu_ref, var_ref,
                  out_ref):
    y = acc_ref[0] + acc_ref[1]
    d = den_ref[0, :, 0:1] + den_ref[1, :, 0:1]
    y = y / (d + 1e-16) + bias_ref[...]
    scale = g_ref[...] * lax.rsqrt(var_ref[...] + 1e-5)
    y = scale * (y - mu_ref[...]) + be_ref[...]
    i = pl.program_id(0)
    rid = lax.broadcasted_iota(jnp.int32, (BLK, 1), 0) + i * BLK
    out_ref[...] = jnp.where(rid < N, jnp.maximum(y, F32_0), F32_0)


def _combine(acc, den, bias, gamma, beta, mean, var):
    return pl.pallas_call(
        _combine_body,
        grid=(NBLK,),
        in_specs=[
            pl.BlockSpec((2, BLK, F), lambda i: (Z, i, Z)),
            pl.BlockSpec((2, BLK, 8), lambda i: (Z, i, Z)),
        ] + [pl.BlockSpec((1, F), lambda i: (Z, Z))] * 5,
        out_specs=pl.BlockSpec((BLK, F), lambda i: (i, Z)),
        out_shape=jax.ShapeDtypeStruct((NP, F), jnp.float32),
    )(acc, den, bias, gamma, beta, mean, var)


def _flash_body(xq_ref, xkv_ref, wq_ref, wk_ref, wv_ref, bq_ref, bk_ref,
                bv_ref, wo_ref, bo_ref, w1_ref, b1_ref, w2_ref, b2_ref,
                w3_ref, b3_ref, out_ref,
                macc, mrow, lrow, psum, pmax):
    qi = pl.program_id(0)
    kj = pl.program_id(1)
    nkv = pl.num_programs(1)

    @pl.when(kj == 0)
    def _():
        macc[...] = jnp.zeros_like(macc)
        mrow[...] = jnp.full_like(mrow, NEG)
        lrow[...] = jnp.zeros_like(lrow)

    q = (jnp.dot(xq_ref[...], wq_ref[...], preferred_element_type=jnp.float32)
         + bq_ref[...])
    k = (jnp.dot(xkv_ref[...], wk_ref[...], preferred_element_type=jnp.float32)
         + bk_ref[...])
    v = (jnp.dot(xkv_ref[...], wv_ref[...], preferred_element_type=jnp.float32)
         + bv_ref[...])
    s = lax.dot_general(q, k, (((1,), (1,)), ((), ())),
                        preferred_element_type=jnp.float32) * 0.125
    cid = lax.broadcasted_iota(jnp.int32, (BLK, BLK), 1) + kj * BLK
    s = jnp.where(cid < N, s, NEG)

    m_prev = mrow[...]
    m_new = jnp.maximum(m_prev, jnp.max(s, axis=1, keepdims=True))
    p = jnp.exp(s - m_new[:, 0:1])
    corr = jnp.exp(m_prev - m_new)
    lrow[...] = lrow[...] * corr + jnp.sum(p, axis=1, keepdims=True)
    macc[...] = (macc[...] * corr[:, 0:F]
                 + jnp.dot(p, v, preferred_element_type=jnp.float32))
    mrow[...] = m_new

    @pl.when(kj == nkv - 1)
    def _():
        attn = macc[...] / lrow[:, 0:1]
        y = xq_ref[...] + jnp.dot(attn, wo_ref[...],
                                  preferred_element_type=jnp.float32) + bo_ref[...]
        rid = lax.broadcasted_iota(jnp.int32, (BLK, 1), 0) + qi * BLK
        rvalid = rid < N
        cur_sum = jnp.sum(jnp.where(rvalid, y, F32_0), axis=0, keepdims=True)
        cur_max = jnp.max(jnp.where(rvalid, y, NEG), axis=0, keepdims=True)

        @pl.when(qi == 0)
        def _():
            psum[0:1, :] = cur_sum
            pmax[0:1, :] = cur_max

        @pl.when(qi > 0)
        def _():
            psum[0:1, :] = psum[0:1, :] + cur_sum
            pmax[0:1, :] = jnp.maximum(pmax[0:1, :], cur_max)

        @pl.when(qi == pl.num_programs(0) - 1)
        def _():
            g = jnp.concatenate(
                [psum[0:1, :] * (1.0 / N), pmax[0:1, :]], axis=1)
            h1 = jnp.maximum(
                jnp.dot(g, w1_ref[...], preferred_element_type=jnp.float32)
                + b1_ref[...], F32_0)
            h2 = jnp.maximum(
                jnp.dot(h1, w2_ref[...], preferred_element_type=jnp.float32)
                + b2_ref[...], F32_0)
            out_ref[...] = (jnp.dot(h2, w3_ref[...],
                                    preferred_element_type=jnp.float32)
                            + b3_ref[...])


def _flash(x3, attn_p, mlp_p):
    wq = attn_p["in_w"][0:F].T
    wk = attn_p["in_w"][F:2 * F].T
    wv = attn_p["in_w"][2 * F:].T
    bq = attn_p["in_b"][0:F].reshape(1, F)
    bk = attn_p["in_b"][F:2 * F].reshape(1, F)
    bv = attn_p["in_b"][2 * F:].reshape(1, F)
    wo = attn_p["out_w"].T
    bo = attn_p["out_b"].reshape(1, F)
    w1, b1 = mlp_p["W1"], mlp_p["b1"].reshape(1, -1)
    w2, b2 = mlp_p["W2"], mlp_p["b2"].reshape(1, -1)
    w3, b3 = mlp_p["W3"], mlp_p["b3"].reshape(1, -1)

    const = lambda shape: pl.BlockSpec(shape, lambda i, j: (Z,) * len(shape))
    return pl.pallas_call(
        _flash_body,
        grid=(NBLK, NBLK),
        in_specs=[
            pl.BlockSpec((BLK, F), lambda i, j: (i, Z)),
            pl.BlockSpec((BLK, F), lambda i, j: (j, Z)),
            const((F, F)), const((F, F)), const((F, F)),
            const((1, F)), const((1, F)), const((1, F)),
            const((F, F)), const((1, F)),
            const((2 * F, 128)), const((1, 128)),
            const((128, 64)), const((1, 64)),
            const((64, 192)), const((1, 192)),
        ],
        out_specs=pl.BlockSpec((1, 192), lambda i, j: (Z, Z)),
        out_shape=jax.ShapeDtypeStruct((1, 192), jnp.float32),
        scratch_shapes=[
            pltpu.VMEM((BLK, F), jnp.float32),
            pltpu.VMEM((BLK, 128), jnp.float32),
            pltpu.VMEM((BLK, 128), jnp.float32),
            pltpu.VMEM((8, F), jnp.float32),
            pltpu.VMEM((8, F), jnp.float32),
        ],
    )(x3, x3, wq, wk, wv, bq, bk, bv, wo, bo, w1, b1, w2, b2, w3, b3)


def _mesh():
    return plsc.VectorSubcoreMesh(core_axis_name="c", subcore_axis_name="s",
                                  num_cores=NC, num_subcores=NS)


def _sc_scatter8(rows8, idx, zrow, K, W):
    @functools.partial(
        pl.kernel,
        out_type=jax.ShapeDtypeStruct((NC, NP, 8), jnp.float32),
        mesh=_mesh(),
        compiler_params=pltpu.CompilerParams(use_tc_tiling_on_sc=False, needs_layout_passes=False),
        scratch_types=[
            pltpu.VMEM((K,), jnp.int32),
            pltpu.VMEM((K, 8), jnp.float32),
            pltpu.VMEM((NROW, 8), jnp.float32),
            pltpu.VMEM_SHARED((NP, 8), jnp.float32),
        ],
    )
    def k(rows_hbm, idx_hbm, zr_hbm, out_hbm, idx_v, rows_v, zrow_v, acc_s):
        c = lax.axis_index("c")
        s = lax.axis_index("s")
        s_row = s * np.int32(NROW)
        pltpu.sync_copy(zr_hbm, zrow_v)
        pltpu.sync_copy(zrow_v, acc_s.at[pl.ds(s_row, NROW)])
        plsc.subcore_barrier()

        tile_base = (c * np.int32(NS) + s) * np.int32(K * W)

        def window(w, carry):
            base = tile_base + w * np.int32(K)
            pltpu.sync_copy(idx_hbm.at[pl.ds(base, K)], idx_v)
            pltpu.sync_copy(rows_hbm.at[pl.ds(base, K), :], rows_v)
            pltpu.sync_copy(rows_v, acc_s.at[idx_v], add=True)
            return carry

        lax.fori_loop(np.int32(0), np.int32(W), window, np.int32(0))
        plsc.subcore_barrier()
        pltpu.sync_copy(acc_s.at[pl.ds(s_row, NROW)],
                        out_hbm.at[c, pl.ds(s_row, NROW), :])

    return k(rows8, idx, zrow)


def _sc_spmm(src_f, dst_f, b_all, a_src, a_dst, c16, h, zeros64):
    @functools.partial(
        pl.kernel,
        out_type=[
            jax.ShapeDtypeStruct((NC, NP, F), jnp.float32),
            jax.ShapeDtypeStruct((EPAD,), jnp.float32),
        ],
        mesh=_mesh(),
        compiler_params=pltpu.CompilerParams(use_tc_tiling_on_sc=False, needs_layout_passes=False),
        scratch_types=[
            pltpu.VMEM((NP,), jnp.float32),
            pltpu.VMEM((NP,), jnp.float32),
            pltpu.VMEM((16,), jnp.float32),
            pltpu.VMEM((KS,), jnp.int32),
            pltpu.VMEM((KS,), jnp.int32),
            pltpu.VMEM((KS,), jnp.float32),
            pltpu.VMEM((KS,), jnp.float32),
            pltpu.VMEM((KS, F), jnp.float32),
            pltpu.VMEM((NROW, F), jnp.float32),
            pltpu.VMEM_SHARED((NP, F), jnp.float32),
            pltpu.SemaphoreType.DMA,
        ],
    )
    def k(src_hbm, dst_hbm, b_hbm, as_hbm, ad_hbm, c_hbm, h_hbm, z_hbm,
          out_hbm, ex_hbm,
          as_v, ad_v, c_v, src_v, dst_v, b_v, ex_v, rows_v, zb_v,
          acc_s, gsem):
        c = lax.axis_index("c")
        s = lax.axis_index("s")
        s_row = s * np.int32(NROW)
        pltpu.sync_copy(as_hbm, as_v)
        pltpu.sync_copy(ad_hbm, ad_v)
        pltpu.sync_copy(c_hbm, c_v)
        pltpu.sync_copy(z_hbm, zb_v)
        pltpu.sync_copy(zb_v, acc_s.at[pl.ds(s_row, NROW)])
        plsc.subcore_barrier()

        tile_base = (c * np.int32(NS) + s) * np.int32(KS * WS)

        def window(w, carry):
            base = tile_base + w * np.int32(KS)
            pltpu.sync_copy(src_hbm.at[pl.ds(base, KS)], src_v)
            pltpu.sync_copy(dst_hbm.at[pl.ds(base, KS)], dst_v)
            pltpu.sync_copy(b_hbm.at[pl.ds(base, KS)], b_v)
            gdma = pltpu.async_copy(h_hbm.at[src_v], rows_v, gsem)
            cvec = c_v[...]
            for g in range(KS // 16):
                si = src_v[pl.ds(g * 16, 16)]
                di = dst_v[pl.ds(g * 16, 16)]
                al = (plsc.load_gather(as_v, [si])
                      + plsc.load_gather(ad_v, [di])
                      + b_v[pl.ds(g * 16, 16)])
                al = jnp.maximum(al, al * np.float32(0.2))
                ex = jnp.exp(al - cvec)
                ex_v[pl.ds(g * 16, 16)] = ex
            gdma.wait()

            def rbody(kk, cc):
                m = plsc.load_gather(ex_v, [jnp.full((16,), 0, jnp.int32) + kk])
                for j in range(F // 16):
                    rows_v[kk, pl.ds(j * 16, 16)] = (
                        rows_v[kk, pl.ds(j * 16, 16)] * m)
                return cc

            lax.fori_loop(np.int32(0), np.int32(KS), rbody, np.int32(0))
            pltpu.sync_copy(rows_v, acc_s.at[dst_v], add=True)
            pltpu.sync_copy(ex_v, ex_hbm.at[pl.ds(base, KS)])
            return carry

        lax.fori_loop(np.int32(0), np.int32(WS), window, np.int32(0))
        plsc.subcore_barrier()
        pltpu.sync_copy(acc_s.at[pl.ds(s_row, NROW)],
                        out_hbm.at[c, pl.ds(s_row, NROW), :])

    return k(src_f, dst_f, b_all, a_src, a_dst, c16, h, zeros64)


def kernel(x, edge_index, edge_attr, params):
    with jax.enable_x64(False):
        return _kernel_impl(x, edge_index, edge_attr, params)


def _kernel_impl(x, edge_index, edge_attr, params):
    src = edge_index[0].astype(jnp.int32)
    dst = edge_index[1].astype(jnp.int32)
    loop = jnp.arange(N, dtype=jnp.int32)
    pad = jnp.zeros((EPAD - E_ALL,), jnp.int32)
    src_f = jnp.concatenate([src, loop, pad])
    dst_f = jnp.concatenate([dst, loop, pad])
    eaT = edge_attr.T
    zeros64 = jnp.zeros((NROW, F), jnp.float32)
    zrow = jnp.zeros((NROW, 8), jnp.float32)
    bpad = jnp.full((EPAD - E_ALL,), NEG, jnp.float32)

    ea_rows8 = _rows8_ea(eaT)
    ea_deg = _sc_scatter8(ea_rows8, dst, zrow, 400, 25)

    x_cur = jnp.zeros((NP, IN_DIM), jnp.float32).at[:N].set(x)
    din = IN_DIM
    for i in range(3):
        gp = params["gat"][i]
        bp = params["bn"][i]
        h, a_vec, amax = _layer_prep(
            x_cur, gp["W"], gp["att_src"].reshape(1, F),
            gp["att_dst"].reshape(1, F), din)
        b_real, bmax1 = _edge_b(eaT, gp["W_edge"], gp["att_edge"].reshape(1, F))
        b_loop, bmax2 = _loop_b(ea_deg, gp["W_edge"],
                                gp["att_edge"].reshape(1, F))
        b_all = jnp.concatenate([b_real[0], b_loop[0, :N], bpad])
        c = amax[0, 0] + amax[0, 1] + jnp.maximum(bmax1[0, 0], bmax2[0, 0])
        c16 = jnp.full((16,), c, jnp.float32)
        acc, ex_all = _sc_spmm(src_f, dst_f, b_all, a_vec[0], a_vec[1], c16,
                               h, zeros64)
        ex_rows8 = _rows8_ex(ex_all)
        den = _sc_scatter8(ex_rows8, dst_f, zrow, 656, 16)
        x_cur = _combine(acc, den, gp["bias"].reshape(1, F),
                         bp["gamma"].reshape(1, F), bp["beta"].reshape(1, F),
                         bp["mean"].reshape(1, F), bp["var"].reshape(1, F))
        din = F

    q = _flash(x_cur, params["attn"], params["mlp"])
    return q.reshape(1, 3, 64)


# speedup vs baseline: 10.4984x; 1.0040x over previous
import functools

import numpy as np

import jax
import jax.numpy as jnp
from jax import lax
from jax.experimental import pallas as pl
from jax.experimental.pallas import tpu as pltpu
from jax.experimental.pallas import tpu_sc as plsc

N = 10000
NP = 10240
EG = 320000
F = 64
IN_DIM = 128
EDGE_DIM = 4

BLK = 512
NBLK = NP // BLK

NEG = np.float32(-1e30)
F32_0 = np.float32(0.0)
Z = np.int32(0)

NC = 2
NS = 16
NTILE = NC * NS
NROW = NP // NS

E_ALL = EG + N
KS = 256
WS = 41
EPAD = NTILE * KS * WS


def _layer_prep_body(x_ref, w_ref, as_ref, ad_ref, h_ref, a_ref, amax_ref):
    i = pl.program_id(0)
    h = jnp.dot(x_ref[...], w_ref[...], preferred_element_type=jnp.float32)
    h_ref[...] = h
    a_s = jnp.sum(h * as_ref[...], axis=1)
    a_d = jnp.sum(h * ad_ref[...], axis=1)
    a_ref[...] = jnp.concatenate([a_s[None, :], a_d[None, :]], axis=0)

    rid = lax.broadcasted_iota(jnp.int32, (BLK,), 0) + i * BLK
    valid = rid < N
    cur = jnp.concatenate(
        [jnp.max(jnp.where(valid, a_s, NEG)).reshape(1, 1),
         jnp.max(jnp.where(valid, a_d, NEG)).reshape(1, 1)], axis=1)

    @pl.when(i == 0)
    def _():
        amax_ref[...] = cur

    @pl.when(i > 0)
    def _():
        amax_ref[...] = jnp.maximum(amax_ref[...], cur)


def _layer_prep(x, W, att_src, att_dst, din):
    return pl.pallas_call(
        _layer_prep_body,
        grid=(NBLK,),
        in_specs=[
            pl.BlockSpec((BLK, din), lambda i: (i, Z)),
            pl.BlockSpec((din, F), lambda i: (Z, Z)),
            pl.BlockSpec((1, F), lambda i: (Z, Z)),
            pl.BlockSpec((1, F), lambda i: (Z, Z)),
        ],
        out_specs=[
            pl.BlockSpec((BLK, F), lambda i: (i, Z)),
            pl.BlockSpec((2, BLK), lambda i: (Z, i)),
            pl.BlockSpec((1, 2), lambda i: (Z, Z)),
        ],
        out_shape=[
            jax.ShapeDtypeStruct((NP, F), jnp.float32),
            jax.ShapeDtypeStruct((2, NP), jnp.float32),
            jax.ShapeDtypeStruct((1, 2), jnp.float32),
        ],
    )(x, W, att_src, att_dst)


EBLK = 2560
NEBLK = EG // EBLK


def _edge_b_body(eaT_ref, we_ref, ae_ref, b_ref, bmax_ref):
    i = pl.program_id(0)
    w4 = jnp.dot(we_ref[...], ae_ref[...].T,
                 preferred_element_type=jnp.float32)
    b = jnp.sum(eaT_ref[...] * w4, axis=0, keepdims=True)
    b_ref[...] = b
    cur = jnp.max(b).reshape(1, 1)

    @pl.when(i == 0)
    def _():
        bmax_ref[...] = cur

    @pl.when(i > 0)
    def _():
        bmax_ref[...] = jnp.maximum(bmax_ref[...], cur)


def _edge_b(eaT, W_edge, att_edge):
    return pl.pallas_call(
        _edge_b_body,
        grid=(NEBLK,),
        in_specs=[
            pl.BlockSpec((EDGE_DIM, EBLK), lambda i: (Z, i)),
            pl.BlockSpec((EDGE_DIM, F), lambda i: (Z, Z)),
            pl.BlockSpec((1, F), lambda i: (Z, Z)),
        ],
        out_specs=[
            pl.BlockSpec((1, EBLK), lambda i: (Z, i)),
            pl.BlockSpec((1, 1), lambda i: (Z, Z)),
        ],
        out_shape=[
            jax.ShapeDtypeStruct((1, EG), jnp.float32),
            jax.ShapeDtypeStruct((1, 1), jnp.float32),
        ],
    )(eaT, W_edge, att_edge)


def _rows8_ea_body(eaT_ref, out_ref):
    ea = eaT_ref[...].T
    ones = jnp.ones((EBLK, 1), jnp.float32)
    zer = jnp.zeros((EBLK, 3), jnp.float32)
    out_ref[...] = jnp.concatenate([ea, ones, zer], axis=1)


def _rows8_ea(eaT):
    return pl.pallas_call(
        _rows8_ea_body,
        grid=(NEBLK,),
        in_specs=[pl.BlockSpec((EDGE_DIM, EBLK), lambda i: (Z, i))],
        out_specs=pl.BlockSpec((EBLK, 8), lambda i: (i, Z)),
        out_shape=jax.ShapeDtypeStruct((EG, 8), jnp.float32),
    )(eaT)


XBLK = 4096
NXBLK = EPAD // XBLK


def _rows8_ex_body(ex_ref, out_ref):
    e = ex_ref[...].reshape(XBLK, 1)
    out_ref[...] = jnp.concatenate(
        [e, jnp.zeros((XBLK, 7), jnp.float32)], axis=1)


def _rows8_ex(ex):
    return pl.pallas_call(
        _rows8_ex_body,
        grid=(NXBLK,),
        in_specs=[pl.BlockSpec((1, XBLK), lambda i: (Z, i))],
        out_specs=pl.BlockSpec((XBLK, 8), lambda i: (i, Z)),
        out_shape=jax.ShapeDtypeStruct((EPAD, 8), jnp.float32),
    )(ex.reshape(1, EPAD))


def _loop_b_body(ed_ref, we_ref, ae_ref, b_ref, bmax_ref):
    i = pl.program_id(0)
    w4 = jnp.dot(we_ref[...], ae_ref[...].T,
                 preferred_element_type=jnp.float32)
    eb = ed_ref[0] + ed_ref[1]
    s = jnp.sum(eb[:, 0:EDGE_DIM] * w4[:, 0].reshape(1, EDGE_DIM), axis=1)
    deg = jnp.maximum(eb[:, EDGE_DIM], 1.0)
    b = (s / deg)[None, :]
    b_ref[...] = b
    cur = jnp.max(b).reshape(1, 1)

    @pl.when(i == 0)
    def _():
        bmax_ref[...] = cur

    @pl.when(i > 0)
    def _():
        bmax_ref[...] = jnp.maximum(bmax_ref[...], cur)


def _loop_b(ea_deg, W_edge, att_edge):
    return pl.pallas_call(
        _loop_b_body,
        grid=(NBLK,),
        in_specs=[
            pl.BlockSpec((2, BLK, 8), lambda i: (Z, i, Z)),
            pl.BlockSpec((EDGE_DIM, F), lambda i: (Z, Z)),
            pl.BlockSpec((1, F), lambda i: (Z, Z)),
        ],
        out_specs=[
            pl.BlockSpec((1, BLK), lambda i: (Z, i)),
            pl.BlockSpec((1, 1), lambda i: (Z, Z)),
        ],
        out_shape=[
            jax.ShapeDtypeStruct((1, NP), jnp.float32),
            jax.ShapeDtypeStruct((1, 1), jnp.float32),
        ],
    )(ea_deg, W_edge, att_edge)


def _combine_body(acc_ref, den_ref, bias_ref, g_ref, be_ref, mu_ref, var_ref,
                  out_ref):
    y = acc_ref[0] + acc_ref[1]
    d = den_ref[0, :, 0:1] + den_ref[1, :, 0:1]
    y = y / (d + 1e-16) + bias_ref[...]
    scale = g_ref[...] * lax.rsqrt(var_ref[...] + 1e-5)
    y = scale * (y - mu_ref[...]) + be_ref[...]
    i = pl.program_id(0)
    rid = lax.broadcasted_iota(jnp.int32, (BLK, 1), 0) + i * BLK
    out_ref[...] = jnp.where(rid < N, jnp.maximum(y, F32_0), F32_0)


def _combine(acc, den, bias, gamma, beta, mean, var):
    return pl.pallas_call(
        _combine_body,
        grid=(NBLK,),
        in_specs=[
            pl.BlockSpec((2, BLK, F), lambda i: (Z, i, Z)),
            pl.BlockSpec((2, BLK, 8), lambda i: (Z, i, Z)),
        ] + [pl.BlockSpec((1, F), lambda i: (Z, Z))] * 5,
        out_specs=pl.BlockSpec((BLK, F), lambda i: (i, Z)),
        out_shape=jax.ShapeDtypeStruct((NP, F), jnp.float32),
    )(acc, den, bias, gamma, beta, mean, var)


def _flash_body(xq_ref, xkv_ref, wq_ref, wk_ref, wv_ref, bq_ref, bk_ref,
                bv_ref, wo_ref, bo_ref, w1_ref, b1_ref, w2_ref, b2_ref,
                w3_ref, b3_ref, out_ref,
                macc, mrow, lrow, psum, pmax):
    qi = pl.program_id(0)
    kj = pl.program_id(1)
    nkv = pl.num_programs(1)

    @pl.when(kj == 0)
    def _():
        macc[...] = jnp.zeros_like(macc)
        mrow[...] = jnp.full_like(mrow, NEG)
        lrow[...] = jnp.zeros_like(lrow)

    q = (jnp.dot(xq_ref[...], wq_ref[...], preferred_element_type=jnp.float32)
         + bq_ref[...])
    k = (jnp.dot(xkv_ref[...], wk_ref[...], preferred_element_type=jnp.float32)
         + bk_ref[...])
    v = (jnp.dot(xkv_ref[...], wv_ref[...], preferred_element_type=jnp.float32)
         + bv_ref[...])
    s = lax.dot_general(q, k, (((1,), (1,)), ((), ())),
                        preferred_element_type=jnp.float32) * 0.125
    cid = lax.broadcasted_iota(jnp.int32, (BLK, BLK), 1) + kj * BLK
    s = jnp.where(cid < N, s, NEG)

    m_prev = mrow[...]
    m_new = jnp.maximum(m_prev, jnp.max(s, axis=1, keepdims=True))
    p = jnp.exp(s - m_new[:, 0:1])
    corr = jnp.exp(m_prev - m_new)
    lrow[...] = lrow[...] * corr + jnp.sum(p, axis=1, keepdims=True)
    macc[...] = (macc[...] * corr[:, 0:F]
                 + jnp.dot(p, v, preferred_element_type=jnp.float32))
    mrow[...] = m_new

    @pl.when(kj == nkv - 1)
    def _():
        attn = macc[...] / lrow[:, 0:1]
        y = xq_ref[...] + jnp.dot(attn, wo_ref[...],
                                  preferred_element_type=jnp.float32) + bo_ref[...]
        rid = lax.broadcasted_iota(jnp.int32, (BLK, 1), 0) + qi * BLK
        rvalid = rid < N
        cur_sum = jnp.sum(jnp.where(rvalid, y, F32_0), axis=0, keepdims=True)
        cur_max = jnp.max(jnp.where(rvalid, y, NEG), axis=0, keepdims=True)

        @pl.when(qi == 0)
        def _():
            psum[0:1, :] = cur_sum
            pmax[0:1, :] = cur_max

        @pl.when(qi > 0)
        def _():
            psum[0:1, :] = psum[0:1, :] + cur_sum
            pmax[0:1, :] = jnp.maximum(pmax[0:1, :], cur_max)

        @pl.when(qi == pl.num_programs(0) - 1)
        def _():
            g = jnp.concatenate(
                [psum[0:1, :] * (1.0 / N), pmax[0:1, :]], axis=1)
            h1 = jnp.maximum(
                jnp.dot(g, w1_ref[...], preferred_element_type=jnp.float32)
                + b1_ref[...], F32_0)
            h2 = jnp.maximum(
                jnp.dot(h1, w2_ref[...], preferred_element_type=jnp.float32)
                + b2_ref[...], F32_0)
            out_ref[...] = (jnp.dot(h2, w3_ref[...],
                                    preferred_element_type=jnp.float32)
                            + b3_ref[...])


def _flash(x3, attn_p, mlp_p):
    wq = attn_p["in_w"][0:F].T
    wk = attn_p["in_w"][F:2 * F].T
    wv = attn_p["in_w"][2 * F:].T
    bq = attn_p["in_b"][0:F].reshape(1, F)
    bk = attn_p["in_b"][F:2 * F].reshape(1, F)
    bv = attn_p["in_b"][2 * F:].reshape(1, F)
    wo = attn_p["out_w"].T
    bo = attn_p["out_b"].reshape(1, F)
    w1, b1 = mlp_p["W1"], mlp_p["b1"].reshape(1, -1)
    w2, b2 = mlp_p["W2"], mlp_p["b2"].reshape(1, -1)
    w3, b3 = mlp_p["W3"], mlp_p["b3"].reshape(1, -1)

    const = lambda shape: pl.BlockSpec(shape, lambda i, j: (Z,) * len(shape))
    return pl.pallas_call(
        _flash_body,
        grid=(NBLK, NBLK),
        in_specs=[
            pl.BlockSpec((BLK, F), lambda i, j: (i, Z)),
            pl.BlockSpec((BLK, F), lambda i, j: (j, Z)),
            const((F, F)), const((F, F)), const((F, F)),
            const((1, F)), const((1, F)), const((1, F)),
            const((F, F)), const((1, F)),
            const((2 * F, 128)), const((1, 128)),
            const((128, 64)), const((1, 64)),
            const((64, 192)), const((1, 192)),
        ],
        out_specs=pl.BlockSpec((1, 192), lambda i, j: (Z, Z)),
        out_shape=jax.ShapeDtypeStruct((1, 192), jnp.float32),
        scratch_shapes=[
            pltpu.VMEM((BLK, F), jnp.float32),
            pltpu.VMEM((BLK, 128), jnp.float32),
            pltpu.VMEM((BLK, 128), jnp.float32),
            pltpu.VMEM((8, F), jnp.float32),
            pltpu.VMEM((8, F), jnp.float32),
        ],
    )(x3, x3, wq, wk, wv, bq, bk, bv, wo, bo, w1, b1, w2, b2, w3, b3)


def _mesh():
    return plsc.VectorSubcoreMesh(core_axis_name="c", subcore_axis_name="s",
                                  num_cores=NC, num_subcores=NS)


def _sc_scatter8(rows8, idx, zrow, K, W):
    @functools.partial(
        pl.kernel,
        out_type=jax.ShapeDtypeStruct((NC, NP, 8), jnp.float32),
        mesh=_mesh(),
        compiler_params=pltpu.CompilerParams(use_tc_tiling_on_sc=False, needs_layout_passes=False),
        scratch_types=[
            pltpu.VMEM((K,), jnp.int32),
            pltpu.VMEM((K, 8), jnp.float32),
            pltpu.VMEM((NROW, 8), jnp.float32),
            pltpu.VMEM_SHARED((NP, 8), jnp.float32),
        ],
    )
    def k(rows_hbm, idx_hbm, zr_hbm, out_hbm, idx_v, rows_v, zrow_v, acc_s):
        c = lax.axis_index("c")
        s = lax.axis_index("s")
        s_row = s * np.int32(NROW)
        pltpu.sync_copy(zr_hbm, zrow_v)
        pltpu.sync_copy(zrow_v, acc_s.at[pl.ds(s_row, NROW)])
        plsc.subcore_barrier()

        tile_base = (c * np.int32(NS) + s) * np.int32(K * W)

        for w in range(W):
            base = tile_base + np.int32(w * K)
            pltpu.sync_copy(idx_hbm.at[pl.ds(base, K)], idx_v)
            pltpu.sync_copy(rows_hbm.at[pl.ds(base, K), :], rows_v)
            pltpu.sync_copy(rows_v, acc_s.at[idx_v], add=True)
        plsc.subcore_barrier()
        pltpu.sync_copy(acc_s.at[pl.ds(s_row, NROW)],
                        out_hbm.at[c, pl.ds(s_row, NROW), :])

    return k(rows8, idx, zrow)


def _sc_spmm(src_f, dst_f, b_all, a_src, a_dst, c16, h, zeros64):
    @functools.partial(
        pl.kernel,
        out_type=[
            jax.ShapeDtypeStruct((NC, NP, F), jnp.float32),
            jax.ShapeDtypeStruct((EPAD,), jnp.float32),
        ],
        mesh=_mesh(),
        compiler_params=pltpu.CompilerParams(use_tc_tiling_on_sc=False, needs_layout_passes=False),
        scratch_types=[
            pltpu.VMEM((NP,), jnp.float32),
            pltpu.VMEM((NP,), jnp.float32),
            pltpu.VMEM((16,), jnp.float32),
            pltpu.VMEM((KS,), jnp.int32),
            pltpu.VMEM((KS,), jnp.int32),
            pltpu.VMEM((KS,), jnp.float32),
            pltpu.VMEM((KS,), jnp.float32),
            pltpu.VMEM((KS, F), jnp.float32),
            pltpu.VMEM((NROW, F), jnp.float32),
            pltpu.VMEM_SHARED((NP, F), jnp.float32),
            pltpu.SemaphoreType.DMA,
        ],
    )
    def k(src_hbm, dst_hbm, b_hbm, as_hbm, ad_hbm, c_hbm, h_hbm, z_hbm,
          out_hbm, ex_hbm,
          as_v, ad_v, c_v, src_v, dst_v, b_v, ex_v, rows_v, zb_v,
          acc_s, gsem):
        c = lax.axis_index("c")
        s = lax.axis_index("s")
        s_row = s * np.int32(NROW)
        pltpu.sync_copy(as_hbm, as_v)
        pltpu.sync_copy(ad_hbm, ad_v)
        pltpu.sync_copy(c_hbm, c_v)
        pltpu.sync_copy(z_hbm, zb_v)
        pltpu.sync_copy(zb_v, acc_s.at[pl.ds(s_row, NROW)])
        plsc.subcore_barrier()

        tile_base = (c * np.int32(NS) + s) * np.int32(KS * WS)

        def window(w, carry):
            base = tile_base + w * np.int32(KS)
            pltpu.sync_copy(src_hbm.at[pl.ds(base, KS)], src_v)
            pltpu.sync_copy(dst_hbm.at[pl.ds(base, KS)], dst_v)
            pltpu.sync_copy(b_hbm.at[pl.ds(base, KS)], b_v)
            gdma = pltpu.async_copy(h_hbm.at[src_v], rows_v, gsem)
            cvec = c_v[...]
            for g in range(KS // 16):
                si = src_v[pl.ds(g * 16, 16)]
                di = dst_v[pl.ds(g * 16, 16)]
                al = (plsc.load_gather(as_v, [si])
                      + plsc.load_gather(ad_v, [di])
                      + b_v[pl.ds(g * 16, 16)])
                al = jnp.maximum(al, al * np.float32(0.2))
                ex = jnp.exp(al - cvec)
                ex_v[pl.ds(g * 16, 16)] = ex
            gdma.wait()

            def rbody(kk, cc):
                m = plsc.load_gather(ex_v, [jnp.full((16,), 0, jnp.int32) + kk])
                for j in range(F // 16):
                    rows_v[kk, pl.ds(j * 16, 16)] = (
                        rows_v[kk, pl.ds(j * 16, 16)] * m)
                return cc

            lax.fori_loop(np.int32(0), np.int32(KS), rbody, np.int32(0))
            pltpu.sync_copy(rows_v, acc_s.at[dst_v], add=True)
            pltpu.sync_copy(ex_v, ex_hbm.at[pl.ds(base, KS)])
            return carry

        lax.fori_loop(np.int32(0), np.int32(WS), window, np.int32(0))
        plsc.subcore_barrier()
        pltpu.sync_copy(acc_s.at[pl.ds(s_row, NROW)],
                        out_hbm.at[c, pl.ds(s_row, NROW), :])

    return k(src_f, dst_f, b_all, a_src, a_dst, c16, h, zeros64)


def kernel(x, edge_index, edge_attr, params):
    with jax.enable_x64(False):
        return _kernel_impl(x, edge_index, edge_attr, params)


def _kernel_impl(x, edge_index, edge_attr, params):
    src = edge_index[0].astype(jnp.int32)
    dst = edge_index[1].astype(jnp.int32)
    loop = jnp.arange(N, dtype=jnp.int32)
    pad = jnp.zeros((EPAD - E_ALL,), jnp.int32)
    src_f = jnp.concatenate([src, loop, pad])
    dst_f = jnp.concatenate([dst, loop, pad])
    eaT = edge_attr.T
    zeros64 = jnp.zeros((NROW, F), jnp.float32)
    zrow = jnp.zeros((NROW, 8), jnp.float32)
    bpad = jnp.full((EPAD - E_ALL,), NEG, jnp.float32)

    ea_rows8 = _rows8_ea(eaT)
    ea_deg = _sc_scatter8(ea_rows8, dst, zrow, 10000, 1)

    x_cur = jnp.zeros((NP, IN_DIM), jnp.float32).at[:N].set(x)
    din = IN_DIM
    for i in range(3):
        gp = params["gat"][i]
        bp = params["bn"][i]
        h, a_vec, amax = _layer_prep(
            x_cur, gp["W"], gp["att_src"].reshape(1, F),
            gp["att_dst"].reshape(1, F), din)
        b_real, bmax1 = _edge_b(eaT, gp["W_edge"], gp["att_edge"].reshape(1, F))
        b_loop, bmax2 = _loop_b(ea_deg, gp["W_edge"],
                                gp["att_edge"].reshape(1, F))
        b_all = jnp.concatenate([b_real[0], b_loop[0, :N], bpad])
        c = amax[0, 0] + amax[0, 1] + jnp.maximum(bmax1[0, 0], bmax2[0, 0])
        c16 = jnp.full((16,), c, jnp.float32)
        acc, ex_all = _sc_spmm(src_f, dst_f, b_all, a_vec[0], a_vec[1], c16,
                               h, zeros64)
        ex_rows8 = _rows8_ex(ex_all)
        den = _sc_scatter8(ex_rows8, dst_f, zrow, 10496, 1)
        x_cur = _combine(acc, den, gp["bias"].reshape(1, F),
                         bp["gamma"].reshape(1, F), bp["beta"].reshape(1, F),
                         bp["mean"].reshape(1, F), bp["var"].reshape(1, F))
        din = F

    q = _flash(x_cur, params["attn"], params["mlp"])
    return q.reshape(1, 3, 64)
